# baseline (speedup 1.0000x reference)
import functools

import jax
import jax.numpy as jnp
from jax import lax
from jax.experimental import pallas as pl
from jax.experimental.pallas import tpu as pltpu
from jax.experimental.pallas import tpu_sc as plsc

D_MODEL = 1024
TOTAL_TOKENS = 32768
NUM_SEGS = 16

_NC = 2
_NS = 16
_NW = _NC * _NS
_TOK_PER_W = TOTAL_TOKENS // _NW
_CHUNK = 32
_NCHUNK = _TOK_PER_W // _CHUNK
_LANES = 16


def _pos_encoding_sc(cs_pad, encoding):
    mesh = plsc.VectorSubcoreMesh(core_axis_name="c", subcore_axis_name="s")

    @functools.partial(
        pl.kernel,
        out_type=jax.ShapeDtypeStruct((TOTAL_TOKENS, D_MODEL), jnp.float32),
        mesh=mesh,
        scratch_types=[
            pltpu.VMEM((32,), jnp.int32),
            pltpu.VMEM((_TOK_PER_W,), jnp.int32),
            pltpu.VMEM((_CHUNK, D_MODEL), jnp.float32),
            pltpu.VMEM((_CHUNK, D_MODEL), jnp.float32),
            pltpu.VMEM((_CHUNK, D_MODEL), jnp.float32),
            pltpu.SemaphoreType.DMA,
            pltpu.SemaphoreType.DMA,
            pltpu.SemaphoreType.DMA,
            pltpu.SemaphoreType.DMA,
            pltpu.SemaphoreType.DMA,
            pltpu.SemaphoreType.DMA,
        ],
    )
    def body(
        cs_hbm, enc_hbm, out_hbm, cs_v, idx_v,
        rows0, rows1, rows2, gs0, gs1, gs2, ws0, ws1, ws2,
    ):
        wid = lax.axis_index("s") * _NC + lax.axis_index("c")
        base = wid * _TOK_PER_W

        pltpu.sync_copy(cs_hbm, cs_v)

        cs_vec = cs_v[pl.ds(0, _LANES)]
        dnums = lax.GatherDimensionNumbers(
            offset_dims=(), collapsed_slice_dims=(0,), start_index_map=(0,)
        )

        def bcast_lane(i):
            return lax.gather(
                cs_vec,
                jnp.full((_LANES, 1), i, jnp.int32),
                dnums,
                (1,),
                mode=lax.GatherScatterMode.PROMISE_IN_BOUNDS,
            )

        bcast = [bcast_lane(i) for i in range(1, NUM_SEGS)]

        def fill_idx(j, _):
            t = lax.iota(jnp.int32, _LANES) + (base + j * _LANES)
            off = jnp.zeros((_LANES,), jnp.int32)
            for b in bcast:
                off = jnp.where(b <= t, b, off)
            idx_v[pl.ds(j * _LANES, _LANES)] = t - off
            return 0

        def fill_chunk(c):
            fill_idx(2 * c, 0)
            fill_idx(2 * c + 1, 0)

        rows = (rows0, rows1, rows2)
        gsems = (gs0, gs1, gs2)
        wsems = (ws0, ws1, ws2)
        nbuf = 3
        lag = 2

        def gather(c, b):
            return pltpu.async_copy(
                enc_hbm.at[idx_v.at[pl.ds(c * _CHUNK, _CHUNK)]], rows[b], gsems[b]
            )

        def write(c, b):
            return pltpu.async_copy(
                rows[b], out_hbm.at[pl.ds(base + c * _CHUNK, _CHUNK)], wsems[b]
            )

        gd = [None] * nbuf
        wd = [None] * nbuf
        for c in range(_NCHUNK + lag):
            if c < _NCHUNK:
                b = c % nbuf
                fill_chunk(c)
                if wd[b] is not None:
                    wd[b].wait()
                gd[b] = gather(c, b)
            if c >= lag:
                cc = c - lag
                b2 = cc % nbuf
                gd[b2].wait()
                wd[b2] = write(cc, b2)
        for c in range(_NCHUNK - nbuf, _NCHUNK):
            wd[c % nbuf].wait()

    return body(cs_pad, encoding)


def kernel(cu_seqlens, encoding):
    cs_pad = jnp.concatenate(
        [
            cu_seqlens.astype(jnp.int32),
            jnp.full((32 - (NUM_SEGS + 1),), TOTAL_TOKENS, jnp.int32),
        ]
    )
    return _pos_encoding_sc(cs_pad, encoding)


# speedup vs baseline: 1.0060x; 1.0060x over previous
import functools

import jax
import jax.numpy as jnp
from jax import lax
from jax.experimental import pallas as pl
from jax.experimental.pallas import tpu as pltpu
from jax.experimental.pallas import tpu_sc as plsc

D_MODEL = 1024
TOTAL_TOKENS = 32768
NUM_SEGS = 16

_NC = 2
_NS = 16
_NW = _NC * _NS
_TOK_PER_W = TOTAL_TOKENS // _NW
_CHUNK = 32
_NCHUNK = _TOK_PER_W // _CHUNK
_LANES = 16


def _pos_encoding_sc(cs_pad, encoding):
    mesh = plsc.VectorSubcoreMesh(core_axis_name="c", subcore_axis_name="s")

    @functools.partial(
        pl.kernel,
        out_type=jax.ShapeDtypeStruct((TOTAL_TOKENS, D_MODEL), jnp.float32),
        mesh=mesh,
        scratch_types=[
            pltpu.VMEM((32,), jnp.int32),
            pltpu.VMEM((_TOK_PER_W,), jnp.int32),
            pltpu.VMEM((_CHUNK, D_MODEL), jnp.float32),
            pltpu.VMEM((_CHUNK, D_MODEL), jnp.float32),
            pltpu.VMEM((_CHUNK, D_MODEL), jnp.float32),
            pltpu.SemaphoreType.DMA,
            pltpu.SemaphoreType.DMA,
            pltpu.SemaphoreType.DMA,
            pltpu.SemaphoreType.DMA,
            pltpu.SemaphoreType.DMA,
            pltpu.SemaphoreType.DMA,
        ],
    )
    def body(
        cs_hbm, enc_hbm, out_hbm, cs_v, idx_v,
        rows0, rows1, rows2, gs0, gs1, gs2, ws0, ws1, ws2,
    ):
        wid = lax.axis_index("s") * _NC + lax.axis_index("c")
        base = wid * _TOK_PER_W

        pltpu.sync_copy(cs_hbm, cs_v)

        cs_vec = cs_v[pl.ds(0, _LANES)]
        dnums = lax.GatherDimensionNumbers(
            offset_dims=(), collapsed_slice_dims=(0,), start_index_map=(0,)
        )

        def bcast_lane(i):
            return lax.gather(
                cs_vec,
                jnp.full((_LANES, 1), i, jnp.int32),
                dnums,
                (1,),
                mode=lax.GatherScatterMode.PROMISE_IN_BOUNDS,
            )

        bcast = [bcast_lane(i) for i in range(1, NUM_SEGS)]

        def fill_idx(j, _):
            t = lax.iota(jnp.int32, _LANES) + (base + j * _LANES)
            off = jnp.zeros((_LANES,), jnp.int32)
            for b in bcast:
                off = jnp.where(b <= t, b, off)
            idx_v[pl.ds(j * _LANES, _LANES)] = t - off
            return 0

        lax.fori_loop(0, _TOK_PER_W // _LANES, fill_idx, 0)

        rows = (rows0, rows1, rows2)
        gsems = (gs0, gs1, gs2)
        wsems = (ws0, ws1, ws2)
        nbuf = 3
        lag = 2

        def gather(c, b):
            return pltpu.async_copy(
                enc_hbm.at[idx_v.at[pl.ds(c * _CHUNK, _CHUNK)]], rows[b], gsems[b]
            )

        def write(c, b):
            return pltpu.async_copy(
                rows[b], out_hbm.at[pl.ds(base + c * _CHUNK, _CHUNK)], wsems[b]
            )

        gd = [None] * nbuf
        wd = [None] * nbuf
        for c in range(_NCHUNK + lag):
            if c < _NCHUNK:
                b = c % nbuf
                if wd[b] is not None:
                    wd[b].wait()
                gd[b] = gather(c, b)
            if c >= lag:
                cc = c - lag
                b2 = cc % nbuf
                gd[b2].wait()
                wd[b2] = write(cc, b2)
        for c in range(_NCHUNK - nbuf, _NCHUNK):
            wd[c % nbuf].wait()

    return body(cs_pad, encoding)


def kernel(cu_seqlens, encoding):
    cs_pad = jnp.concatenate(
        [
            cu_seqlens.astype(jnp.int32),
            jnp.full((32 - (NUM_SEGS + 1),), TOTAL_TOKENS, jnp.int32),
        ]
    )
    return _pos_encoding_sc(cs_pad, encoding)
